# Initial kernel scaffold; baseline (speedup 1.0000x reference)
import jax
import jax.numpy as jnp
from jax.experimental import pallas as pl


def kernel(x, edge_index, edge_attr, cond, params):
    raise NotImplementedError("write your pallas kernel here")


# speedup vs baseline: 2.3332x; 2.3332x over previous
import functools

import jax
import jax.numpy as jnp
from jax import lax
from jax.experimental import pallas as pl
from jax.experimental.pallas import tpu as pltpu
from jax.experimental.pallas import tpu_sc as plsc

N = 10000
E = 320000
D = 128
ED = 16

NCORES = 2
NSUB = 16
NTILES = NCORES * NSUB
EPT = E // NTILES
CHUNK = 80
NCH = EPT // CHUNK
NPAD = 10240
NPT = NPAD // NSUB

_mesh = plsc.VectorSubcoreMesh(core_axis_name="c", subcore_axis_name="s")


def _linear_body(x_ref, w_ref, b_ref, o_ref):
    o_ref[:] = jnp.dot(x_ref[:], w_ref[:],
                       preferred_element_type=jnp.float32,
                       precision=lax.Precision.HIGHEST) + b_ref[:]


def _linear(x, w, b, block_rows):
    n, k = x.shape
    m = w.shape[1]
    return pl.pallas_call(
        _linear_body,
        grid=(n // block_rows,),
        in_specs=[pl.BlockSpec((block_rows, k), lambda i: (i, 0)),
                  pl.BlockSpec((k, m), lambda i: (0, 0)),
                  pl.BlockSpec((1, m), lambda i: (0, 0))],
        out_specs=pl.BlockSpec((block_rows, m), lambda i: (i, 0)),
        out_shape=jax.ShapeDtypeStruct((n, m), jnp.float32),
    )(x, w, b.reshape(1, m))


def _edge_mm_body(ea_ref, w_ref, b_ref, o0, o1, o2, o3):
    r = jnp.dot(ea_ref[:], w_ref[:],
                preferred_element_type=jnp.float32,
                       precision=lax.Precision.HIGHEST) + b_ref[:]
    o0[:] = r[:, 0 * D:1 * D]
    o1[:] = r[:, 1 * D:2 * D]
    o2[:] = r[:, 2 * D:3 * D]
    o3[:] = r[:, 3 * D:4 * D]


def _edge_mm(ea, w_cat, b_cat):
    BR = 4000
    return pl.pallas_call(
        _edge_mm_body,
        grid=(E // BR,),
        in_specs=[pl.BlockSpec((BR, ED), lambda i: (i, 0)),
                  pl.BlockSpec((ED, 4 * D), lambda i: (0, 0)),
                  pl.BlockSpec((1, 4 * D), lambda i: (0, 0))],
        out_specs=[pl.BlockSpec((BR, D), lambda i: (i, 0))] * 4,
        out_shape=[jax.ShapeDtypeStruct((E, D), jnp.float32)] * 4,
    )(ea, w_cat, b_cat.reshape(1, 4 * D))


def _mlp_body(x_ref, ag_ref, w1_ref, b1_ref, w2_ref, b2_ref, o_ref):
    h = x_ref[:] + ag_ref[0] + ag_ref[1]
    a = jnp.maximum(jnp.dot(h, w1_ref[:],
                            preferred_element_type=jnp.float32,
                       precision=lax.Precision.HIGHEST) + b1_ref[:],
                    0.0)
    o = jnp.dot(a, w2_ref[:], preferred_element_type=jnp.float32,
                       precision=lax.Precision.HIGHEST) + b2_ref[:]
    o_ref[:] = jnp.maximum(o, 0.0)


def _mlp(x, ag, w1, b1, w2, b2):
    BR = 2000
    return pl.pallas_call(
        _mlp_body,
        grid=(N // BR,),
        in_specs=[pl.BlockSpec((BR, D), lambda i: (i, 0)),
                  pl.BlockSpec((2, BR, D), lambda i: (0, i, 0)),
                  pl.BlockSpec((D, D), lambda i: (0, 0)),
                  pl.BlockSpec((1, D), lambda i: (0, 0)),
                  pl.BlockSpec((D, D), lambda i: (0, 0)),
                  pl.BlockSpec((1, D), lambda i: (0, 0))],
        out_specs=pl.BlockSpec((BR, D), lambda i: (i, 0)),
        out_shape=jax.ShapeDtypeStruct((N, D), jnp.float32),
    )(x, ag, w1, b1.reshape(1, D), w2, b2.reshape(1, D))


def _pq_body(x_ref, w_ref, p_ref, q_ref):
    r = jnp.dot(x_ref[:], w_ref[:], preferred_element_type=jnp.float32,
                       precision=lax.Precision.HIGHEST)
    p_ref[:] = r[:, :D]
    q_ref[:] = r[:, D:]


def _pq(x, w_pq):
    BR = 2000
    return pl.pallas_call(
        _pq_body,
        grid=(N // BR,),
        in_specs=[pl.BlockSpec((BR, D), lambda i: (i, 0)),
                  pl.BlockSpec((D, 2 * D), lambda i: (0, 0))],
        out_specs=[pl.BlockSpec((BR, D), lambda i: (i, 0))] * 2,
        out_shape=[jax.ShapeDtypeStruct((N, D), jnp.float32)] * 2,
    )(x, w_pq)


def _out_body(z_ref, w_ref, b_ref, o_ref):
    o_ref[:] = jnp.dot(z_ref[:], w_ref[:],
                       preferred_element_type=jnp.float32,
                       precision=lax.Precision.HIGHEST) + b_ref[:]


def _out_mm(z, w2, b2):
    BR = 4000
    m = w2.shape[1]
    return pl.pallas_call(
        _out_body,
        grid=(E // BR,),
        in_specs=[pl.BlockSpec((BR, D), lambda i: (i, 0)),
                  pl.BlockSpec((D, m), lambda i: (0, 0)),
                  pl.BlockSpec((1, m), lambda i: (0, 0))],
        out_specs=pl.BlockSpec((BR, m), lambda i: (i, 0)),
        out_shape=jax.ShapeDtypeStruct((E, m), jnp.float32),
    )(z, w2, b2.reshape(1, m))


def _msg_body(e_hbm, src_hbm, dst_hbm, x_hbm, out_hbm,
              idx_s, idx_d, e_v, xg_v, zbuf, aggr_sh, sem):
    cid = lax.axis_index("c")
    sid = lax.axis_index("s")
    wid = cid * NSUB + sid

    def _zrow(r, c):
        for j in range(8):
            zbuf[r, pl.ds(j * 16, 16)] = jnp.zeros((16,), jnp.float32)
        return c
    lax.fori_loop(0, 128, _zrow, 0)
    for k in range(NPT // 128):
        pltpu.sync_copy(zbuf, aggr_sh.at[pl.ds(sid * NPT + k * 128, 128)])
    plsc.subcore_barrier()

    def _chunk(c, carry):
        base = wid * EPT + c * CHUNK
        pltpu.sync_copy(src_hbm.at[pl.ds(base, CHUNK)], idx_s)
        gat = pltpu.async_copy(x_hbm.at[idx_s], xg_v, sem)
        pltpu.sync_copy(dst_hbm.at[pl.ds(base, CHUNK)], idx_d)
        pltpu.sync_copy(e_hbm.at[pl.ds(base, CHUNK)], e_v)
        gat.wait()

        def _row(r, cc):
            for j in range(8):
                s = pl.ds(j * 16, 16)
                e_v[r, s] = jnp.maximum(e_v[r, s] + xg_v[r, s], 0.0)
            return cc
        lax.fori_loop(0, CHUNK, _row, 0)
        pltpu.sync_copy(e_v, aggr_sh.at[idx_d], add=True)
        return carry
    lax.fori_loop(0, NCH, _chunk, 0)

    plsc.subcore_barrier()
    rows = pl.ds(sid * NPT, NPT)
    pltpu.sync_copy(aggr_sh.at[rows], out_hbm.at[cid].at[rows])


@functools.partial(
    pl.kernel,
    out_type=jax.ShapeDtypeStruct((NCORES, NPAD, D), jnp.float32),
    mesh=_mesh,
    scratch_types=[
        pltpu.VMEM((CHUNK,), jnp.int32),
        pltpu.VMEM((CHUNK,), jnp.int32),
        pltpu.VMEM((CHUNK, D), jnp.float32),
        pltpu.VMEM((CHUNK, D), jnp.float32),
        pltpu.VMEM((128, D), jnp.float32),
        pltpu.VMEM_SHARED((NPAD, D), jnp.float32),
        pltpu.SemaphoreType.DMA,
    ],
)
def _msg(e_hbm, src_hbm, dst_hbm, x_hbm, out_hbm,
         idx_s, idx_d, e_v, xg_v, zbuf, aggr_sh, sem):
    _msg_body(e_hbm, src_hbm, dst_hbm, x_hbm, out_hbm,
              idx_s, idx_d, e_v, xg_v, zbuf, aggr_sh, sem)


def _fin_body(r_hbm, src_hbm, dst_hbm, p_hbm, q_hbm, z_hbm,
              idx_s, idx_d, r_v, pg_v, qg_v, sem_p, sem_q):
    cid = lax.axis_index("c")
    sid = lax.axis_index("s")
    wid = cid * NSUB + sid

    def _chunk(c, carry):
        base = wid * EPT + c * CHUNK
        pltpu.sync_copy(src_hbm.at[pl.ds(base, CHUNK)], idx_s)
        gp = pltpu.async_copy(p_hbm.at[idx_s], pg_v, sem_p)
        pltpu.sync_copy(dst_hbm.at[pl.ds(base, CHUNK)], idx_d)
        gq = pltpu.async_copy(q_hbm.at[idx_d], qg_v, sem_q)
        pltpu.sync_copy(r_hbm.at[pl.ds(base, CHUNK)], r_v)
        gp.wait()
        gq.wait()

        def _row(rr, cc):
            for j in range(8):
                s = pl.ds(j * 16, 16)
                r_v[rr, s] = jnp.maximum(
                    r_v[rr, s] + pg_v[rr, s] + qg_v[rr, s], 0.0)
            return cc
        lax.fori_loop(0, CHUNK, _row, 0)
        pltpu.sync_copy(r_v, z_hbm.at[pl.ds(base, CHUNK)])
        return carry
    lax.fori_loop(0, NCH, _chunk, 0)


@functools.partial(
    pl.kernel,
    out_type=jax.ShapeDtypeStruct((E, D), jnp.float32),
    mesh=_mesh,
    scratch_types=[
        pltpu.VMEM((CHUNK,), jnp.int32),
        pltpu.VMEM((CHUNK,), jnp.int32),
        pltpu.VMEM((CHUNK, D), jnp.float32),
        pltpu.VMEM((CHUNK, D), jnp.float32),
        pltpu.VMEM((CHUNK, D), jnp.float32),
        pltpu.SemaphoreType.DMA,
        pltpu.SemaphoreType.DMA,
    ],
)
def _fin(r_hbm, src_hbm, dst_hbm, p_hbm, q_hbm, z_hbm,
         idx_s, idx_d, r_v, pg_v, qg_v, sem_p, sem_q):
    _fin_body(r_hbm, src_hbm, dst_hbm, p_hbm, q_hbm, z_hbm,
              idx_s, idx_d, r_v, pg_v, qg_v, sem_p, sem_q)


def kernel(x, edge_index, edge_attr, cond, params):
    p = params
    src = edge_index[0]
    dst = edge_index[1]

    gamma = cond @ p['gamma_W'] + p['gamma_b']
    beta = cond @ p['beta_W'] + p['beta_b']
    wa = p['cls1_W'][:D]
    wb = p['cls1_W'][D:2 * D]
    wc = p['cls1_W'][2 * D:]
    w_cat = jnp.concatenate(
        [gamma[:, None] * p['convs'][l]['edge_W'] for l in range(3)]
        + [gamma[:, None] * wc], axis=1)
    b_cat = jnp.concatenate(
        [beta @ p['convs'][l]['edge_W'] + p['convs'][l]['edge_b']
         for l in range(3)]
        + [beta @ wc + p['cls1_b']], axis=0)

    xc = _linear(x, p['lin1_W'], p['lin1_b'], 2000)
    e1, e2, e3, r4 = _edge_mm(edge_attr, w_cat, b_cat)

    for l, e_l in enumerate((e1, e2, e3)):
        cv = p['convs'][l]
        ag = _msg(e_l, src, dst, xc)
        xc = _mlp(xc, ag, cv['nn1_W'], cv['nn1_b'],
                  cv['nn2_W'], cv['nn2_b'])

    pt, qt = _pq(xc, jnp.concatenate([wa, wb], axis=1))
    z = _fin(r4, src, dst, pt, qt)
    return _out_mm(z, p['cls2_W'], p['cls2_b'])


# speedup vs baseline: 3.2962x; 1.4128x over previous
import functools

import numpy as np

import jax
import jax.numpy as jnp
from jax import lax
from jax.experimental import pallas as pl
from jax.experimental.pallas import tpu as pltpu
from jax.experimental.pallas import tpu_sc as plsc

N = 10000
E = 320000
D = 128
ED = 16

NCORES = 2
NSUB = 16
NTILES = NCORES * NSUB
EPT = E // NTILES
CHUNK = 80
NCH = EPT // CHUNK
NPAD = 10240
NPT = NPAD // NSUB
DW = D // 2

_mesh = plsc.VectorSubcoreMesh(core_axis_name="c", subcore_axis_name="s")

_LO_COLS = np.arange(128).reshape(4, 2, 16)[:, 0, :].reshape(-1)
_HI_COLS = np.arange(128).reshape(4, 2, 16)[:, 1, :].reshape(-1)
_MASK_HI = -65536
_ROUND = 0x8000


def _lohi(w):
    return jnp.concatenate([w[:, _LO_COLS], w[:, _HI_COLS]], axis=1)


def _pack_tc(lo, hi):
    il = lax.shift_right_logical(
        lax.bitcast_convert_type(lo, jnp.int32) + _ROUND, 16)
    ih = (lax.bitcast_convert_type(hi, jnp.int32) + _ROUND) & _MASK_HI
    return il | ih


def _unpack_tc(w):
    lo = lax.bitcast_convert_type(lax.shift_left(w, 16), jnp.float32)
    hi = lax.bitcast_convert_type(w & _MASK_HI, jnp.float32)
    return lo, hi


def _dot(a, b):
    return jnp.dot(a, b, preferred_element_type=jnp.float32,
                   precision=lax.Precision.HIGHEST)


def _linear_body(x_ref, w_ref, b_ref, o_ref):
    o_ref[:] = _dot(x_ref[:], w_ref[:]) + b_ref[:]


def _linear(x, w, b):
    BR = 2000
    return pl.pallas_call(
        _linear_body,
        grid=(N // BR,),
        in_specs=[pl.BlockSpec((BR, D), lambda i: (i, 0)),
                  pl.BlockSpec((D, D), lambda i: (0, 0)),
                  pl.BlockSpec((1, D), lambda i: (0, 0))],
        out_specs=pl.BlockSpec((BR, D), lambda i: (i, 0)),
        out_shape=jax.ShapeDtypeStruct((N, D), jnp.float32),
    )(x, w, b.reshape(1, D))


def _edge_mm_body(ea_ref, w_ref, b_ref, *outs):
    r = _dot(ea_ref[:], w_ref[:]) + b_ref[:]
    for j, o in enumerate(outs):
        o[:] = _pack_tc(r[:, j * D:j * D + DW], r[:, j * D + DW:(j + 1) * D])


def _edge_mm(ea, w_ord, b_ord, nout):
    BR = 4000
    m = nout * D
    return pl.pallas_call(
        _edge_mm_body,
        grid=(E // BR,),
        in_specs=[pl.BlockSpec((BR, ED), lambda i: (i, 0)),
                  pl.BlockSpec((ED, m), lambda i: (0, 0)),
                  pl.BlockSpec((1, m), lambda i: (0, 0))],
        out_specs=[pl.BlockSpec((BR, DW), lambda i: (i, 0))] * nout,
        out_shape=[jax.ShapeDtypeStruct((E, DW), jnp.int32)] * nout,
    )(ea, w_ord, b_ord.reshape(1, m))


def _mlp_body(x_ref, ag_ref, w1_ref, b1_ref, w2_ref, b2_ref, o_ref):
    h = x_ref[:] + ag_ref[0] + ag_ref[1]
    a = jnp.maximum(_dot(h, w1_ref[:]) + b1_ref[:], 0.0)
    o_ref[:] = jnp.maximum(_dot(a, w2_ref[:]) + b2_ref[:], 0.0)


def _mlp(x, ag, w1, b1, w2, b2):
    BR = 2000
    return pl.pallas_call(
        _mlp_body,
        grid=(N // BR,),
        in_specs=[pl.BlockSpec((BR, D), lambda i: (i, 0)),
                  pl.BlockSpec((2, BR, D), lambda i: (0, i, 0)),
                  pl.BlockSpec((D, D), lambda i: (0, 0)),
                  pl.BlockSpec((1, D), lambda i: (0, 0)),
                  pl.BlockSpec((D, D), lambda i: (0, 0)),
                  pl.BlockSpec((1, D), lambda i: (0, 0))],
        out_specs=pl.BlockSpec((BR, D), lambda i: (i, 0)),
        out_shape=jax.ShapeDtypeStruct((N, D), jnp.float32),
    )(x, ag, w1, b1.reshape(1, D), w2, b2.reshape(1, D))


def _pq_body(x_ref, w_ref, p_ref, q_ref):
    r = _dot(x_ref[:], w_ref[:])
    p_ref[:] = r[:, :D]
    q_ref[:] = r[:, D:]


def _pq(x, w_pq):
    BR = 2000
    return pl.pallas_call(
        _pq_body,
        grid=(N // BR,),
        in_specs=[pl.BlockSpec((BR, D), lambda i: (i, 0)),
                  pl.BlockSpec((D, 2 * D), lambda i: (0, 0))],
        out_specs=[pl.BlockSpec((BR, D), lambda i: (i, 0))] * 2,
        out_shape=[jax.ShapeDtypeStruct((N, D), jnp.float32)] * 2,
    )(x, w_pq)


def _out_body(z_ref, wl_ref, wh_ref, b_ref, o_ref):
    zl, zh = _unpack_tc(z_ref[:])
    o_ref[:] = (_dot(zl, wl_ref[:]) + _dot(zh, wh_ref[:]) + b_ref[:])


def _out_mm(z, w2, b2):
    BR = 4000
    m = w2.shape[1]
    wl = w2[_LO_COLS, :]
    wh = w2[_HI_COLS, :]
    return pl.pallas_call(
        _out_body,
        grid=(E // BR,),
        in_specs=[pl.BlockSpec((BR, DW), lambda i: (i, 0)),
                  pl.BlockSpec((DW, m), lambda i: (0, 0)),
                  pl.BlockSpec((DW, m), lambda i: (0, 0)),
                  pl.BlockSpec((1, m), lambda i: (0, 0))],
        out_specs=pl.BlockSpec((BR, m), lambda i: (i, 0)),
        out_shape=jax.ShapeDtypeStruct((E, m), jnp.float32),
    )(z, wl, wh, b2.reshape(1, m))


def _unpack_sc(w):
    lo = lax.bitcast_convert_type(lax.shift_left(w, 16), jnp.float32)
    hi = lax.bitcast_convert_type(w & _MASK_HI, jnp.float32)
    return lo, hi


def _msg_body(e_hbm, src_hbm, dst_hbm, x_hbm, out_hbm,
              idxs, idxd, e_v, xg_v, aggr_sh,
              sem_is, sem_id, sem_g, sem_e):
    cid = lax.axis_index("c")
    sid = lax.axis_index("s")
    wid = cid * NSUB + sid

    def _zrow(r, c):
        for j in range(8):
            xg_v[0][r, pl.ds(j * 16, 16)] = jnp.zeros((16,), jnp.float32)
        return c
    lax.fori_loop(0, CHUNK, _zrow, 0)
    for k in range(NPT // CHUNK):
        pltpu.sync_copy(xg_v[0],
                        aggr_sh.at[pl.ds(sid * NPT + k * CHUNK, CHUNK)])
    plsc.subcore_barrier()

    def _base(i):
        return wid * EPT + i * CHUNK

    def _fire_idx(i, b):
        pltpu.async_copy(src_hbm.at[pl.ds(_base(i), CHUNK)], idxs[b],
                         sem_is[b])
        pltpu.async_copy(dst_hbm.at[pl.ds(_base(i), CHUNK)], idxd[b],
                         sem_id[b])

    def _wait_idx(i, b):
        pltpu.make_async_copy(src_hbm.at[pl.ds(_base(i), CHUNK)], idxs[b],
                              sem_is[b]).wait()
        pltpu.make_async_copy(dst_hbm.at[pl.ds(_base(i), CHUNK)], idxd[b],
                              sem_id[b]).wait()

    def _fire_data(i, b):
        pltpu.async_copy(x_hbm.at[idxs[b]], xg_v[b], sem_g[b])
        pltpu.async_copy(e_hbm.at[pl.ds(_base(i), CHUNK)], e_v[b], sem_e[b])

    def _wait_data(i, b):
        pltpu.make_async_copy(x_hbm.at[idxs[b]], xg_v[b], sem_g[b]).wait()
        pltpu.make_async_copy(e_hbm.at[pl.ds(_base(i), CHUNK)], e_v[b],
                              sem_e[b]).wait()

    def _compute(b):
        def _row(r, cc):
            for j in range(4):
                el, eh = _unpack_sc(e_v[b][r, pl.ds(j * 16, 16)])
                sl = pl.ds(j * 32, 16)
                sh = pl.ds(j * 32 + 16, 16)
                xg_v[b][r, sl] = jnp.maximum(el + xg_v[b][r, sl], 0.0)
                xg_v[b][r, sh] = jnp.maximum(eh + xg_v[b][r, sh], 0.0)
            return cc
        lax.fori_loop(0, CHUNK, _row, 0)

    def _step(i, b, do_fire_idx2):
        _wait_idx(i + 1, 1 - b)
        _fire_data(i + 1, 1 - b)
        _wait_data(i, b)
        _compute(b)
        pltpu.sync_copy(xg_v[b], aggr_sh.at[idxd[b]], add=True)

        @pl.when(do_fire_idx2)
        def _():
            _fire_idx(i + 2, b)

    _fire_idx(0, 0)
    _fire_idx(1, 1)
    _wait_idx(0, 0)
    _fire_data(0, 0)

    def _giter(g, c):
        _step(2 * g, 0, jnp.bool_(True))
        _step(2 * g + 1, 1, g < (NCH - 1) // 2 - 1)
        return c
    lax.fori_loop(0, (NCH - 1) // 2, _giter, 0)

    _wait_data(NCH - 1, 0)
    _compute(0)
    pltpu.sync_copy(xg_v[0], aggr_sh.at[idxd[0]], add=True)

    plsc.subcore_barrier()
    rows = pl.ds(sid * NPT, NPT)
    pltpu.sync_copy(aggr_sh.at[rows], out_hbm.at[cid].at[rows])


@functools.partial(
    pl.kernel,
    out_type=jax.ShapeDtypeStruct((NCORES, NPAD, D), jnp.float32),
    mesh=_mesh,
    scratch_types=[
        pltpu.VMEM((CHUNK,), jnp.int32),
        pltpu.VMEM((CHUNK,), jnp.int32),
        pltpu.VMEM((CHUNK,), jnp.int32),
        pltpu.VMEM((CHUNK,), jnp.int32),
        pltpu.VMEM((CHUNK, DW), jnp.int32),
        pltpu.VMEM((CHUNK, DW), jnp.int32),
        pltpu.VMEM((CHUNK, D), jnp.float32),
        pltpu.VMEM((CHUNK, D), jnp.float32),
        pltpu.VMEM_SHARED((NPAD, D), jnp.float32),
        pltpu.SemaphoreType.DMA,
        pltpu.SemaphoreType.DMA,
        pltpu.SemaphoreType.DMA,
        pltpu.SemaphoreType.DMA,
        pltpu.SemaphoreType.DMA,
        pltpu.SemaphoreType.DMA,
        pltpu.SemaphoreType.DMA,
        pltpu.SemaphoreType.DMA,
    ],
)
def _msg(e_hbm, src_hbm, dst_hbm, x_hbm, out_hbm,
         is0, is1, id0, id1, ev0, ev1, xg0, xg1, aggr_sh,
         si0, si1, sd0, sd1, sg0, sg1, se0, se1):
    _msg_body(e_hbm, src_hbm, dst_hbm, x_hbm, out_hbm,
              (is0, is1), (id0, id1), (ev0, ev1), (xg0, xg1),
              aggr_sh, (si0, si1), (sd0, sd1), (sg0, sg1), (se0, se1))


def _fin_body(r_hbm, src_hbm, dst_hbm, p_hbm, q_hbm, z_hbm,
              idxs, idxd, r_v, pg_v, qg_v, z_v,
              sem_is, sem_id, sem_p, sem_q, sem_r):
    cid = lax.axis_index("c")
    sid = lax.axis_index("s")
    wid = cid * NSUB + sid

    def _base(i):
        return wid * EPT + i * CHUNK

    def _fire_idx(i, b):
        pltpu.async_copy(src_hbm.at[pl.ds(_base(i), CHUNK)], idxs[b],
                         sem_is[b])
        pltpu.async_copy(dst_hbm.at[pl.ds(_base(i), CHUNK)], idxd[b],
                         sem_id[b])

    def _wait_idx(i, b):
        pltpu.make_async_copy(src_hbm.at[pl.ds(_base(i), CHUNK)], idxs[b],
                              sem_is[b]).wait()
        pltpu.make_async_copy(dst_hbm.at[pl.ds(_base(i), CHUNK)], idxd[b],
                              sem_id[b]).wait()

    def _fire_data(i, b):
        pltpu.async_copy(p_hbm.at[idxs[b]], pg_v[b], sem_p[b])
        pltpu.async_copy(q_hbm.at[idxd[b]], qg_v[b], sem_q[b])
        pltpu.async_copy(r_hbm.at[pl.ds(_base(i), CHUNK)], r_v[b], sem_r[b])

    def _wait_data(i, b):
        pltpu.make_async_copy(p_hbm.at[idxs[b]], pg_v[b], sem_p[b]).wait()
        pltpu.make_async_copy(q_hbm.at[idxd[b]], qg_v[b], sem_q[b]).wait()
        pltpu.make_async_copy(r_hbm.at[pl.ds(_base(i), CHUNK)], r_v[b],
                              sem_r[b]).wait()

    def _compute(b):
        def _row(r, cc):
            for j in range(4):
                s = pl.ds(j * 16, 16)
                sl = pl.ds(j * 32, 16)
                sh = pl.ds(j * 32 + 16, 16)
                rl, rh = _unpack_sc(r_v[b][r, s])
                zl = jnp.maximum(rl + pg_v[b][r, sl] + qg_v[b][r, sl], 0.0)
                zh = jnp.maximum(rh + pg_v[b][r, sh] + qg_v[b][r, sh], 0.0)
                il = lax.shift_right_logical(
                    lax.bitcast_convert_type(zl, jnp.int32) + _ROUND, 16)
                ih = (lax.bitcast_convert_type(zh, jnp.int32)
                      + _ROUND) & _MASK_HI
                z_v[r, s] = il | ih
            return cc
        lax.fori_loop(0, CHUNK, _row, 0)

    def _step(i, b, do_fire_idx2):
        _wait_idx(i + 1, 1 - b)
        _fire_data(i + 1, 1 - b)
        _wait_data(i, b)
        _compute(b)
        pltpu.sync_copy(z_v, z_hbm.at[pl.ds(_base(i), CHUNK)])

        @pl.when(do_fire_idx2)
        def _():
            _fire_idx(i + 2, b)

    _fire_idx(0, 0)
    _fire_idx(1, 1)
    _wait_idx(0, 0)
    _fire_data(0, 0)

    def _giter(g, c):
        _step(2 * g, 0, jnp.bool_(True))
        _step(2 * g + 1, 1, g < (NCH - 1) // 2 - 1)
        return c
    lax.fori_loop(0, (NCH - 1) // 2, _giter, 0)

    _wait_data(NCH - 1, 0)
    _compute(0)
    pltpu.sync_copy(z_v, z_hbm.at[pl.ds(_base(NCH - 1), CHUNK)])


@functools.partial(
    pl.kernel,
    out_type=jax.ShapeDtypeStruct((E, DW), jnp.int32),
    mesh=_mesh,
    scratch_types=[
        pltpu.VMEM((CHUNK,), jnp.int32),
        pltpu.VMEM((CHUNK,), jnp.int32),
        pltpu.VMEM((CHUNK,), jnp.int32),
        pltpu.VMEM((CHUNK,), jnp.int32),
        pltpu.VMEM((CHUNK, DW), jnp.int32),
        pltpu.VMEM((CHUNK, DW), jnp.int32),
        pltpu.VMEM((CHUNK, D), jnp.float32),
        pltpu.VMEM((CHUNK, D), jnp.float32),
        pltpu.VMEM((CHUNK, D), jnp.float32),
        pltpu.VMEM((CHUNK, D), jnp.float32),
        pltpu.VMEM((CHUNK, DW), jnp.int32),
        pltpu.SemaphoreType.DMA,
        pltpu.SemaphoreType.DMA,
        pltpu.SemaphoreType.DMA,
        pltpu.SemaphoreType.DMA,
        pltpu.SemaphoreType.DMA,
        pltpu.SemaphoreType.DMA,
        pltpu.SemaphoreType.DMA,
        pltpu.SemaphoreType.DMA,
        pltpu.SemaphoreType.DMA,
        pltpu.SemaphoreType.DMA,
    ],
)
def _fin(r_hbm, src_hbm, dst_hbm, p_hbm, q_hbm, z_hbm,
         is0, is1, id0, id1, rv0, rv1, pg0, pg1, qg0, qg1, zv,
         si0, si1, sd0, sd1, sp0, sp1, sq0, sq1, sr0, sr1):
    _fin_body(r_hbm, src_hbm, dst_hbm, p_hbm, q_hbm, z_hbm,
              (is0, is1), (id0, id1), (rv0, rv1), (pg0, pg1), (qg0, qg1),
              zv, (si0, si1), (sd0, sd1), (sp0, sp1), (sq0, sq1), (sr0, sr1))


def kernel(x, edge_index, edge_attr, cond, params):
    p = params
    src = edge_index[0]
    dst = edge_index[1]

    gamma = cond @ p['gamma_W'] + p['gamma_b']
    beta = cond @ p['beta_W'] + p['beta_b']
    wa = p['cls1_W'][:D]
    wb = p['cls1_W'][D:2 * D]
    wc = p['cls1_W'][2 * D:]
    ew = [gamma[:, None] * p['convs'][l]['edge_W'] for l in range(3)]
    ew.append(gamma[:, None] * wc)
    eb = [beta @ p['convs'][l]['edge_W'] + p['convs'][l]['edge_b']
          for l in range(3)]
    eb.append(beta @ wc + p['cls1_b'])
    ew = [_lohi(w) for w in ew]
    eb = [_lohi(b.reshape(1, D)).reshape(-1) for b in eb]

    xc = _linear(x, p['lin1_W'], p['lin1_b'])
    (e1,) = _edge_mm(edge_attr, ew[0], eb[0], 1)
    e2, e3, r4 = _edge_mm(edge_attr, jnp.concatenate(ew[1:], axis=1),
                          jnp.concatenate(eb[1:], axis=0), 3)

    for l, e_l in enumerate((e1, e2, e3)):
        cv = p['convs'][l]
        ag = _msg(e_l, src, dst, xc)
        xc = _mlp(xc, ag, cv['nn1_W'], cv['nn1_b'],
                  cv['nn2_W'], cv['nn2_b'])

    pt, qt = _pq(xc, jnp.concatenate([wa, wb], axis=1))
    z = _fin(r4, src, dst, pt, qt)
    return _out_mm(z, p['cls2_W'], p['cls2_b'])


# speedup vs baseline: 4.0201x; 1.2196x over previous
import functools

import numpy as np

import jax
import jax.numpy as jnp
from jax import lax
from jax.experimental import pallas as pl
from jax.experimental.pallas import tpu as pltpu
from jax.experimental.pallas import tpu_sc as plsc

N = 10000
E = 320000
D = 128
ED = 16

NCORES = 2
NSUB = 16
NTILES = NCORES * NSUB
EPT = E // NTILES
CHUNK = 80
NCH = EPT // CHUNK
NPAD = 10240
NPT = NPAD // NSUB
DW = D // 2

_mesh = plsc.VectorSubcoreMesh(core_axis_name="c", subcore_axis_name="s")

_LO_COLS = np.arange(128).reshape(4, 2, 16)[:, 0, :].reshape(-1)
_HI_COLS = np.arange(128).reshape(4, 2, 16)[:, 1, :].reshape(-1)
_MASK_HI = -65536
_ROUND = 0x8000


def _lohi(w):
    return jnp.concatenate([w[:, _LO_COLS], w[:, _HI_COLS]], axis=1)


def _pack_tc(lo, hi):
    il = lax.shift_right_logical(
        lax.bitcast_convert_type(lo, jnp.int32) + _ROUND, 16)
    ih = (lax.bitcast_convert_type(hi, jnp.int32) + _ROUND) & _MASK_HI
    return il | ih


def _unpack_tc(w):
    lo = lax.bitcast_convert_type(lax.shift_left(w, 16), jnp.float32)
    hi = lax.bitcast_convert_type(w & _MASK_HI, jnp.float32)
    return lo, hi


def _dot(a, b):
    return jnp.dot(a, b, preferred_element_type=jnp.float32,
                   precision=lax.Precision.HIGHEST)


def _dot_fast(a, b):
    return jnp.dot(a, b, preferred_element_type=jnp.float32,
                   precision=lax.Precision.DEFAULT)


def _linear_body(x_ref, w_ref, b_ref, o_ref):
    o_ref[:] = _dot(x_ref[:], w_ref[:]) + b_ref[:]


def _linear(x, w, b):
    BR = 2000
    return pl.pallas_call(
        _linear_body,
        grid=(N // BR,),
        in_specs=[pl.BlockSpec((BR, D), lambda i: (i, 0)),
                  pl.BlockSpec((D, D), lambda i: (0, 0)),
                  pl.BlockSpec((1, D), lambda i: (0, 0))],
        out_specs=pl.BlockSpec((BR, D), lambda i: (i, 0)),
        out_shape=jax.ShapeDtypeStruct((N, D), jnp.float32),
    )(x, w, b.reshape(1, D))


def _pack_tc_trunc(lo, hi):
    il = lax.shift_right_logical(lax.bitcast_convert_type(lo, jnp.int32), 16)
    ih = lax.bitcast_convert_type(hi, jnp.int32) & _MASK_HI
    return il | ih


def _edge_mm_body(ea_ref, w_ref, b_ref, *outs):
    r = _dot_fast(ea_ref[:], w_ref[:]) + b_ref[:]
    for j, o in enumerate(outs):
        o[:] = _pack_tc_trunc(r[:, j * D:j * D + DW],
                              r[:, j * D + DW:(j + 1) * D])


def _edge_mm(ea, w_ord, b_ord, nout):
    BR = 4000
    m = nout * D
    return pl.pallas_call(
        _edge_mm_body,
        grid=(E // BR,),
        in_specs=[pl.BlockSpec((BR, ED), lambda i: (i, 0)),
                  pl.BlockSpec((ED, m), lambda i: (0, 0)),
                  pl.BlockSpec((1, m), lambda i: (0, 0))],
        out_specs=[pl.BlockSpec((BR, DW), lambda i: (i, 0))] * nout,
        out_shape=[jax.ShapeDtypeStruct((E, DW), jnp.int32)] * nout,
    )(ea, w_ord, b_ord.reshape(1, m))


def _mlp_body(x_ref, ag_ref, w1_ref, b1_ref, w2_ref, b2_ref, o_ref):
    h = x_ref[:] + ag_ref[0] + ag_ref[1]
    a = jnp.maximum(_dot(h, w1_ref[:]) + b1_ref[:], 0.0)
    o_ref[:] = jnp.maximum(_dot(a, w2_ref[:]) + b2_ref[:], 0.0)


def _mlp(x, ag, w1, b1, w2, b2):
    BR = 2000
    return pl.pallas_call(
        _mlp_body,
        grid=(N // BR,),
        in_specs=[pl.BlockSpec((BR, D), lambda i: (i, 0)),
                  pl.BlockSpec((2, BR, D), lambda i: (0, i, 0)),
                  pl.BlockSpec((D, D), lambda i: (0, 0)),
                  pl.BlockSpec((1, D), lambda i: (0, 0)),
                  pl.BlockSpec((D, D), lambda i: (0, 0)),
                  pl.BlockSpec((1, D), lambda i: (0, 0))],
        out_specs=pl.BlockSpec((BR, D), lambda i: (i, 0)),
        out_shape=jax.ShapeDtypeStruct((N, D), jnp.float32),
    )(x, ag, w1, b1.reshape(1, D), w2, b2.reshape(1, D))


def _pq_body(x_ref, w_ref, p_ref, q_ref):
    r = _dot(x_ref[:], w_ref[:])
    p_ref[:] = r[:, :D]
    q_ref[:] = r[:, D:]


def _pq(x, w_pq):
    BR = 2000
    return pl.pallas_call(
        _pq_body,
        grid=(N // BR,),
        in_specs=[pl.BlockSpec((BR, D), lambda i: (i, 0)),
                  pl.BlockSpec((D, 2 * D), lambda i: (0, 0))],
        out_specs=[pl.BlockSpec((BR, D), lambda i: (i, 0))] * 2,
        out_shape=[jax.ShapeDtypeStruct((N, D), jnp.float32)] * 2,
    )(x, w_pq)


def _out_body(z_ref, wl_ref, wh_ref, b_ref, o_ref):
    zl, zh = _unpack_tc(z_ref[:])
    o_ref[:] = (_dot_fast(zl, wl_ref[:]) + _dot_fast(zh, wh_ref[:])
                + b_ref[:])


def _out_mm(z, w2, b2):
    BR = 4000
    m = w2.shape[1]
    wl = w2[_LO_COLS, :]
    wh = w2[_HI_COLS, :]
    return pl.pallas_call(
        _out_body,
        grid=(E // BR,),
        in_specs=[pl.BlockSpec((BR, DW), lambda i: (i, 0)),
                  pl.BlockSpec((DW, m), lambda i: (0, 0)),
                  pl.BlockSpec((DW, m), lambda i: (0, 0)),
                  pl.BlockSpec((1, m), lambda i: (0, 0))],
        out_specs=pl.BlockSpec((BR, m), lambda i: (i, 0)),
        out_shape=jax.ShapeDtypeStruct((E, m), jnp.float32),
    )(z, wl, wh, b2.reshape(1, m))


def _unpack_sc(w):
    lo = lax.bitcast_convert_type(lax.shift_left(w, 16), jnp.float32)
    hi = lax.bitcast_convert_type(w & _MASK_HI, jnp.float32)
    return lo, hi


def _msg_body(e_hbm, src_hbm, dst_hbm, x_hbm, out_hbm,
              idxs, idxd, e_v, xg_v, aggr_sh,
              sem_is, sem_id, sem_g, sem_e):
    cid = lax.axis_index("c")
    sid = lax.axis_index("s")
    wid = cid * NSUB + sid

    def _zrow(r, c):
        for j in range(8):
            xg_v[0][r, pl.ds(j * 16, 16)] = jnp.zeros((16,), jnp.float32)
        return c
    lax.fori_loop(0, CHUNK, _zrow, 0)
    for k in range(NPT // CHUNK):
        pltpu.sync_copy(xg_v[0],
                        aggr_sh.at[pl.ds(sid * NPT + k * CHUNK, CHUNK)])
    plsc.subcore_barrier()

    def _base(i):
        return wid * EPT + i * CHUNK

    def _fire_idx(i, b):
        pltpu.async_copy(src_hbm.at[pl.ds(_base(i), CHUNK)], idxs[b],
                         sem_is[b])
        pltpu.async_copy(dst_hbm.at[pl.ds(_base(i), CHUNK)], idxd[b],
                         sem_id[b])

    def _wait_idx(i, b):
        pltpu.make_async_copy(src_hbm.at[pl.ds(_base(i), CHUNK)], idxs[b],
                              sem_is[b]).wait()
        pltpu.make_async_copy(dst_hbm.at[pl.ds(_base(i), CHUNK)], idxd[b],
                              sem_id[b]).wait()

    def _fire_data(i, b):
        pltpu.async_copy(x_hbm.at[idxs[b]], xg_v[b], sem_g[b])
        pltpu.async_copy(e_hbm.at[pl.ds(_base(i), CHUNK)], e_v[b], sem_e[b])

    def _wait_data(i, b):
        pltpu.make_async_copy(x_hbm.at[idxs[b]], xg_v[b], sem_g[b]).wait()
        pltpu.make_async_copy(e_hbm.at[pl.ds(_base(i), CHUNK)], e_v[b],
                              sem_e[b]).wait()

    def _compute(b):
        def _row(r, cc):
            for j in range(4):
                el, eh = _unpack_sc(e_v[b][r, pl.ds(j * 16, 16)])
                sl = pl.ds(j * 32, 16)
                sh = pl.ds(j * 32 + 16, 16)
                xg_v[b][r, sl] = jnp.maximum(el + xg_v[b][r, sl], 0.0)
                xg_v[b][r, sh] = jnp.maximum(eh + xg_v[b][r, sh], 0.0)
            return cc
        lax.fori_loop(0, CHUNK, _row, 0)

    def _step(i, b, do_fire_idx2):
        _wait_idx(i + 1, 1 - b)
        _fire_data(i + 1, 1 - b)
        _wait_data(i, b)
        _compute(b)
        pltpu.sync_copy(xg_v[b], aggr_sh.at[idxd[b]], add=True)

        @pl.when(do_fire_idx2)
        def _():
            _fire_idx(i + 2, b)

    _fire_idx(0, 0)
    _fire_idx(1, 1)
    _wait_idx(0, 0)
    _fire_data(0, 0)

    def _giter(g, c):
        _step(2 * g, 0, jnp.bool_(True))
        _step(2 * g + 1, 1, g < (NCH - 1) // 2 - 1)
        return c
    lax.fori_loop(0, (NCH - 1) // 2, _giter, 0)

    _wait_data(NCH - 1, 0)
    _compute(0)
    pltpu.sync_copy(xg_v[0], aggr_sh.at[idxd[0]], add=True)

    plsc.subcore_barrier()
    rows = pl.ds(sid * NPT, NPT)
    pltpu.sync_copy(aggr_sh.at[rows], out_hbm.at[cid].at[rows])


@functools.partial(
    pl.kernel,
    out_type=jax.ShapeDtypeStruct((NCORES, NPAD, D), jnp.float32),
    mesh=_mesh,
    scratch_types=[
        pltpu.VMEM((CHUNK,), jnp.int32),
        pltpu.VMEM((CHUNK,), jnp.int32),
        pltpu.VMEM((CHUNK,), jnp.int32),
        pltpu.VMEM((CHUNK,), jnp.int32),
        pltpu.VMEM((CHUNK, DW), jnp.int32),
        pltpu.VMEM((CHUNK, DW), jnp.int32),
        pltpu.VMEM((CHUNK, D), jnp.float32),
        pltpu.VMEM((CHUNK, D), jnp.float32),
        pltpu.VMEM_SHARED((NPAD, D), jnp.float32),
        pltpu.SemaphoreType.DMA,
        pltpu.SemaphoreType.DMA,
        pltpu.SemaphoreType.DMA,
        pltpu.SemaphoreType.DMA,
        pltpu.SemaphoreType.DMA,
        pltpu.SemaphoreType.DMA,
        pltpu.SemaphoreType.DMA,
        pltpu.SemaphoreType.DMA,
    ],
)
def _msg(e_hbm, src_hbm, dst_hbm, x_hbm, out_hbm,
         is0, is1, id0, id1, ev0, ev1, xg0, xg1, aggr_sh,
         si0, si1, sd0, sd1, sg0, sg1, se0, se1):
    _msg_body(e_hbm, src_hbm, dst_hbm, x_hbm, out_hbm,
              (is0, is1), (id0, id1), (ev0, ev1), (xg0, xg1),
              aggr_sh, (si0, si1), (sd0, sd1), (sg0, sg1), (se0, se1))


def _fin_body(r_hbm, src_hbm, dst_hbm, p_hbm, q_hbm, z_hbm,
              idxs, idxd, r_v, pg_v, qg_v, z_v,
              sem_is, sem_id, sem_p, sem_q, sem_r):
    cid = lax.axis_index("c")
    sid = lax.axis_index("s")
    wid = cid * NSUB + sid

    def _base(i):
        return wid * EPT + i * CHUNK

    def _fire_idx(i, b):
        pltpu.async_copy(src_hbm.at[pl.ds(_base(i), CHUNK)], idxs[b],
                         sem_is[b])
        pltpu.async_copy(dst_hbm.at[pl.ds(_base(i), CHUNK)], idxd[b],
                         sem_id[b])

    def _wait_idx(i, b):
        pltpu.make_async_copy(src_hbm.at[pl.ds(_base(i), CHUNK)], idxs[b],
                              sem_is[b]).wait()
        pltpu.make_async_copy(dst_hbm.at[pl.ds(_base(i), CHUNK)], idxd[b],
                              sem_id[b]).wait()

    def _fire_data(i, b):
        pltpu.async_copy(p_hbm.at[idxs[b]], pg_v[b], sem_p[b])
        pltpu.async_copy(q_hbm.at[idxd[b]], qg_v[b], sem_q[b])
        pltpu.async_copy(r_hbm.at[pl.ds(_base(i), CHUNK)], r_v[b], sem_r[b])

    def _wait_data(i, b):
        pltpu.make_async_copy(p_hbm.at[idxs[b]], pg_v[b], sem_p[b]).wait()
        pltpu.make_async_copy(q_hbm.at[idxd[b]], qg_v[b], sem_q[b]).wait()
        pltpu.make_async_copy(r_hbm.at[pl.ds(_base(i), CHUNK)], r_v[b],
                              sem_r[b]).wait()

    def _compute(b):
        def _row(r, cc):
            for j in range(4):
                s = pl.ds(j * 16, 16)
                sl = pl.ds(j * 32, 16)
                sh = pl.ds(j * 32 + 16, 16)
                rl, rh = _unpack_sc(r_v[b][r, s])
                zl = jnp.maximum(rl + pg_v[b][r, sl] + qg_v[b][r, sl], 0.0)
                zh = jnp.maximum(rh + pg_v[b][r, sh] + qg_v[b][r, sh], 0.0)
                il = lax.shift_right_logical(
                    lax.bitcast_convert_type(zl, jnp.int32) + _ROUND, 16)
                ih = (lax.bitcast_convert_type(zh, jnp.int32)
                      + _ROUND) & _MASK_HI
                z_v[r, s] = il | ih
            return cc
        lax.fori_loop(0, CHUNK, _row, 0)

    def _step(i, b, do_fire_idx2):
        _wait_idx(i + 1, 1 - b)
        _fire_data(i + 1, 1 - b)
        _wait_data(i, b)
        _compute(b)
        pltpu.sync_copy(z_v, z_hbm.at[pl.ds(_base(i), CHUNK)])

        @pl.when(do_fire_idx2)
        def _():
            _fire_idx(i + 2, b)

    _fire_idx(0, 0)
    _fire_idx(1, 1)
    _wait_idx(0, 0)
    _fire_data(0, 0)

    def _giter(g, c):
        _step(2 * g, 0, jnp.bool_(True))
        _step(2 * g + 1, 1, g < (NCH - 1) // 2 - 1)
        return c
    lax.fori_loop(0, (NCH - 1) // 2, _giter, 0)

    _wait_data(NCH - 1, 0)
    _compute(0)
    pltpu.sync_copy(z_v, z_hbm.at[pl.ds(_base(NCH - 1), CHUNK)])


@functools.partial(
    pl.kernel,
    out_type=jax.ShapeDtypeStruct((E, DW), jnp.int32),
    mesh=_mesh,
    scratch_types=[
        pltpu.VMEM((CHUNK,), jnp.int32),
        pltpu.VMEM((CHUNK,), jnp.int32),
        pltpu.VMEM((CHUNK,), jnp.int32),
        pltpu.VMEM((CHUNK,), jnp.int32),
        pltpu.VMEM((CHUNK, DW), jnp.int32),
        pltpu.VMEM((CHUNK, DW), jnp.int32),
        pltpu.VMEM((CHUNK, D), jnp.float32),
        pltpu.VMEM((CHUNK, D), jnp.float32),
        pltpu.VMEM((CHUNK, D), jnp.float32),
        pltpu.VMEM((CHUNK, D), jnp.float32),
        pltpu.VMEM((CHUNK, DW), jnp.int32),
        pltpu.SemaphoreType.DMA,
        pltpu.SemaphoreType.DMA,
        pltpu.SemaphoreType.DMA,
        pltpu.SemaphoreType.DMA,
        pltpu.SemaphoreType.DMA,
        pltpu.SemaphoreType.DMA,
        pltpu.SemaphoreType.DMA,
        pltpu.SemaphoreType.DMA,
        pltpu.SemaphoreType.DMA,
        pltpu.SemaphoreType.DMA,
    ],
)
def _fin(r_hbm, src_hbm, dst_hbm, p_hbm, q_hbm, z_hbm,
         is0, is1, id0, id1, rv0, rv1, pg0, pg1, qg0, qg1, zv,
         si0, si1, sd0, sd1, sp0, sp1, sq0, sq1, sr0, sr1):
    _fin_body(r_hbm, src_hbm, dst_hbm, p_hbm, q_hbm, z_hbm,
              (is0, is1), (id0, id1), (rv0, rv1), (pg0, pg1), (qg0, qg1),
              zv, (si0, si1), (sd0, sd1), (sp0, sp1), (sq0, sq1), (sr0, sr1))


def kernel(x, edge_index, edge_attr, cond, params):
    p = params
    src = edge_index[0]
    dst = edge_index[1]

    gamma = cond @ p['gamma_W'] + p['gamma_b']
    beta = cond @ p['beta_W'] + p['beta_b']
    wa = p['cls1_W'][:D]
    wb = p['cls1_W'][D:2 * D]
    wc = p['cls1_W'][2 * D:]
    ew = [gamma[:, None] * p['convs'][l]['edge_W'] for l in range(3)]
    ew.append(gamma[:, None] * wc)
    eb = [beta @ p['convs'][l]['edge_W'] + p['convs'][l]['edge_b']
          for l in range(3)]
    eb.append(beta @ wc + p['cls1_b'])
    ew = [_lohi(w) for w in ew]
    eb = [_lohi(b.reshape(1, D)).reshape(-1) for b in eb]

    xc = _linear(x, p['lin1_W'], p['lin1_b'])
    (e1,) = _edge_mm(edge_attr, ew[0], eb[0], 1)
    e2, e3, r4 = _edge_mm(edge_attr, jnp.concatenate(ew[1:], axis=1),
                          jnp.concatenate(eb[1:], axis=0), 3)

    for l, e_l in enumerate((e1, e2, e3)):
        cv = p['convs'][l]
        ag = _msg(e_l, src, dst, xc)
        xc = _mlp(xc, ag, cv['nn1_W'], cv['nn1_b'],
                  cv['nn2_W'], cv['nn2_b'])

    pt, qt = _pq(xc, jnp.concatenate([wa, wb], axis=1))
    z = _fin(r4, src, dst, pt, qt)
    return _out_mm(z, p['cls2_W'], p['cls2_b'])
